# baseline (speedup 1.0000x reference)
import functools

import jax
import jax.numpy as jnp
from jax import lax
from jax.experimental import pallas as pl
from jax.experimental.pallas import tpu as pltpu
from jax.experimental.pallas import tpu_sc as plsc

N = 10000
E = 320000
D = 128
ROW_BLK = 1000

NW = 32
NSUB = 16
NPAD = 10240
TPR = NPAD // NSUB
GCHUNK = 128
NCH = E // GCHUNK
NBUF = 3
NBUFH = 2

_mesh = plsc.VectorSubcoreMesh(core_axis_name="c", subcore_axis_name="s")
_CP = pltpu.CompilerParams(needs_layout_passes=False)


def _ids():
    cid = lax.axis_index("c")
    sid = lax.axis_index("s")
    return cid, sid, sid * 2 + cid


def _ntrips(wid):
    return (NCH - wid + NW - 1) // NW


def _deg_body(dst_hbm, degp_hbm, sdeg, onesv, idxv, zb,
              isem, ssem):
    cid, sid, wid = _ids()
    ones = jnp.ones((16,), jnp.float32)
    zrow = jnp.zeros((16,), jnp.float32)
    for j in range(GCHUNK // 16):
        onesv[pl.ds(j * 16, 16)] = ones

    def _zb(i, _):
        zb[pl.ds(i * 16, 16)] = zrow
        return 0

    lax.fori_loop(0, TPR // 16, _zb, 0)
    pltpu.sync_copy(zb, sdeg.at[pl.ds(sid * TPR, TPR)])
    plsc.subcore_barrier()

    ntrips = _ntrips(wid)
    ngrp = (ntrips + NBUF - 1) // NBUF

    def _grp(g, _):
        base = g * NBUF
        for q in range(NBUF):
            t = base + q
            c = wid + t * NW

            @pl.when(t < ntrips)
            def _(q=q, t=t, c=c):
                @pl.when(t >= NBUF)
                def _():
                    pltpu.make_async_copy(onesv, sdeg.at[idxv.at[q]],
                                          ssem.at[q]).wait()
                pltpu.async_copy(dst_hbm.at[pl.ds(c * GCHUNK, GCHUNK)],
                                 idxv.at[q], isem.at[q])
        for q in range(NBUF):
            t = base + q

            @pl.when(t < ntrips)
            def _(q=q):
                pltpu.make_async_copy(dst_hbm.at[pl.ds(0, GCHUNK)],
                                      idxv.at[q], isem.at[q]).wait()
                pltpu.async_copy(onesv, sdeg.at[idxv.at[q]], ssem.at[q],
                                 add=True)
        return 0

    lax.fori_loop(0, ngrp, _grp, 0)
    for q in range(NBUF):
        @pl.when(q < ntrips)
        def _(q=q):
            pltpu.make_async_copy(onesv, sdeg.at[idxv.at[q]],
                                  ssem.at[q]).wait()
    plsc.subcore_barrier()

    pltpu.sync_copy(sdeg.at[pl.ds(sid * TPR, TPR)],
                    degp_hbm.at[cid, pl.ds(sid * TPR, TPR)])


def _sc_deg(dst):
    f = pl.kernel(
        _deg_body,
        compiler_params=_CP,
        out_type=jax.ShapeDtypeStruct((2, NPAD), jnp.float32),
        mesh=_mesh,
        scratch_types=[
            pltpu.VMEM_SHARED((NPAD,), jnp.float32),
            pltpu.VMEM((GCHUNK,), jnp.float32),
            pltpu.VMEM((NBUF, GCHUNK), jnp.int32),
            pltpu.VMEM((TPR,), jnp.float32),
            pltpu.SemaphoreType.DMA((NBUF,)),
            pltpu.SemaphoreType.DMA((NBUF,)),
        ],
    )
    return f(dst)


def _hop_body(u_hbm, ei_hbm, part_hbm,
              sacc, gbuf, eidx, zbuf, isem, gsem, ssem):
    cid, sid, wid = _ids()
    zrow = jnp.zeros((16,), jnp.float32)

    def _zb(i, _):
        for j in range(D // 16):
            zbuf[i, pl.ds(j * 16, 16)] = zrow
        return 0

    lax.fori_loop(0, 32, _zb, 0)
    for i in range(TPR // 32):
        pltpu.sync_copy(zbuf,
                        sacc.at[pl.ds(sid * TPR + i * 32, 32), :])
    plsc.subcore_barrier()

    ntrips = _ntrips(wid)
    ngrp = (ntrips + NBUFH - 1) // NBUFH

    def _grp(g, _):
        base = g * NBUFH
        for q in range(NBUFH):
            t = base + q
            c = wid + t * NW

            @pl.when(t < ntrips)
            def _(q=q, t=t, c=c):
                @pl.when(t >= NBUFH)
                def _():
                    pltpu.make_async_copy(gbuf.at[q],
                                          sacc.at[eidx.at[q, 1]],
                                          ssem.at[q]).wait()
                pltpu.async_copy(ei_hbm.at[:, pl.ds(c * GCHUNK, GCHUNK)],
                                 eidx.at[q], isem.at[q])
        for q in range(NBUFH):
            t = base + q

            @pl.when(t < ntrips)
            def _(q=q):
                pltpu.make_async_copy(ei_hbm.at[:, pl.ds(0, GCHUNK)],
                                      eidx.at[q], isem.at[q]).wait()
                pltpu.async_copy(u_hbm.at[eidx.at[q, 0]], gbuf.at[q],
                                 gsem.at[q])
        for q in range(NBUFH):
            t = base + q

            @pl.when(t < ntrips)
            def _(q=q):
                pltpu.make_async_copy(u_hbm.at[eidx.at[q, 0]], gbuf.at[q],
                                      gsem.at[q]).wait()
                pltpu.async_copy(gbuf.at[q], sacc.at[eidx.at[q, 1]],
                                 ssem.at[q], add=True)
        return 0

    lax.fori_loop(0, ngrp, _grp, 0)
    for q in range(NBUFH):
        @pl.when(q < ntrips)
        def _(q=q):
            pltpu.make_async_copy(gbuf.at[q], sacc.at[eidx.at[q, 1]],
                                  ssem.at[q]).wait()
    plsc.subcore_barrier()

    for i in range(TPR // GCHUNK):
        r = sid * TPR + i * GCHUNK
        pltpu.sync_copy(sacc.at[pl.ds(r, GCHUNK), :],
                        part_hbm.at[cid, pl.ds(r, GCHUNK), :])


def _sc_hop(u, ei):
    f = pl.kernel(
        _hop_body,
        compiler_params=_CP,
        out_type=jax.ShapeDtypeStruct((2, NPAD, D), jnp.float32),
        mesh=_mesh,
        scratch_types=[
            pltpu.VMEM_SHARED((NPAD, D), jnp.float32),
            pltpu.VMEM((NBUFH, GCHUNK, D), jnp.float32),
            pltpu.VMEM((NBUFH, 2, GCHUNK), jnp.int32),
            pltpu.VMEM((32, D), jnp.float32),
            pltpu.SemaphoreType.DMA((NBUFH,)),
            pltpu.SemaphoreType.DMA((NBUFH,)),
            pltpu.SemaphoreType.DMA((NBUFH,)),
        ],
    )
    return f(u, ei)


def _pre_body(x_ref, w_ref, b_ref, d0_ref, d1_ref,
              h_ref, u_ref, dis_ref, dis2_ref):
    deg = d0_ref[...] + d1_ref[...]
    dis = jnp.where(deg > 0, lax.rsqrt(jnp.maximum(deg, 1.0)), 0.0)
    dis_ref[...] = dis
    dis2_ref[...] = dis * dis
    h = jnp.maximum(
        jnp.dot(x_ref[...], w_ref[...], preferred_element_type=jnp.float32)
        + b_ref[...], 0.0)
    h_ref[...] = h
    u_ref[...] = h * dis


def _hopcomb_body(p0_ref, p1_ref, dis2_ref, a_ref, u_ref):
    s = p0_ref[0] + p1_ref[0]
    a_ref[...] = s
    u_ref[...] = s * dis2_ref[...]


def _ln(acc, g, bt):
    mu = jnp.mean(acc, axis=-1, keepdims=True)
    var = jnp.mean((acc - mu) ** 2, axis=-1, keepdims=True)
    return (acc - mu) * lax.rsqrt(var + 1e-5) * g + bt


def _combine_body(h0_ref, a1_ref, a2_ref, p30_ref, p31_ref, dis_ref, wk_ref,
                  b_ref, g_ref, bt_ref, h_ref, u_ref):
    d = dis_ref[...]
    acc = jnp.dot(h0_ref[...], wk_ref[0], preferred_element_type=jnp.float32)
    acc += jnp.dot(a1_ref[...] * d, wk_ref[1], preferred_element_type=jnp.float32)
    acc += jnp.dot(a2_ref[...] * d, wk_ref[2], preferred_element_type=jnp.float32)
    a3 = p30_ref[0] + p31_ref[0]
    acc += jnp.dot(a3 * d, wk_ref[3], preferred_element_type=jnp.float32)
    acc = jnp.maximum(acc + b_ref[...], 0.0)
    y = _ln(acc, g_ref[...], bt_ref[...])
    h_ref[...] = y
    u_ref[...] = y * d


def _final_body(h0_ref, a1_ref, a2_ref, p30_ref, p31_ref, dis_ref, wk_ref,
                b_ref, g_ref, bt_ref, wp1_ref, bp1_ref, wp2_ref, bp2_ref,
                o_ref):
    d = dis_ref[...]
    acc = jnp.dot(h0_ref[...], wk_ref[0], preferred_element_type=jnp.float32)
    acc += jnp.dot(a1_ref[...] * d, wk_ref[1], preferred_element_type=jnp.float32)
    acc += jnp.dot(a2_ref[...] * d, wk_ref[2], preferred_element_type=jnp.float32)
    a3 = p30_ref[0] + p31_ref[0]
    acc += jnp.dot(a3 * d, wk_ref[3], preferred_element_type=jnp.float32)
    acc = jnp.maximum(acc + b_ref[...], 0.0)
    y = _ln(acc, g_ref[...], bt_ref[...])
    z = jnp.maximum(
        jnp.dot(y, wp1_ref[...], preferred_element_type=jnp.float32)
        + bp1_ref[...], 0.0)
    o_ref[...] = jnp.dot(z, wp2_ref[...], preferred_element_type=jnp.float32) \
        + bp2_ref[...]


def _row_spec(blk, width):
    return pl.BlockSpec((blk, width), lambda i: (i, 0))


def _part_spec(core):
    return pl.BlockSpec((1, ROW_BLK, D), lambda i, core=core: (core, i, 0))


def _full_spec(shape):
    nd = len(shape)
    return pl.BlockSpec(shape, lambda i: (0,) * nd)


_GRID = (N // ROW_BLK,)


def _pre(x, w, b, deg0, deg1):
    return pl.pallas_call(
        _pre_body,
        grid=_GRID,
        in_specs=[_row_spec(ROW_BLK, D), _full_spec((D, D)),
                  _full_spec((1, D)), _row_spec(ROW_BLK, 1),
                  _row_spec(ROW_BLK, 1)],
        out_specs=[_row_spec(ROW_BLK, D), _row_spec(ROW_BLK, D),
                   _row_spec(ROW_BLK, 1), _row_spec(ROW_BLK, 1)],
        out_shape=[jax.ShapeDtypeStruct((N, D), jnp.float32),
                   jax.ShapeDtypeStruct((N, D), jnp.float32),
                   jax.ShapeDtypeStruct((N, 1), jnp.float32),
                   jax.ShapeDtypeStruct((N, 1), jnp.float32)],
    )(x, w, b.reshape(1, D), deg0, deg1)


def _hopcomb(part, dis2):
    return pl.pallas_call(
        _hopcomb_body,
        grid=_GRID,
        in_specs=[_part_spec(0), _part_spec(1), _row_spec(ROW_BLK, 1)],
        out_specs=[_row_spec(ROW_BLK, D), _row_spec(ROW_BLK, D)],
        out_shape=[jax.ShapeDtypeStruct((N, D), jnp.float32)] * 2,
    )(part, part, dis2)


def _combine(h0, a1, a2, part3, dis, wk, b, g, bt):
    return pl.pallas_call(
        _combine_body,
        grid=_GRID,
        in_specs=[_row_spec(ROW_BLK, D)] * 3
        + [_part_spec(0), _part_spec(1)]
        + [_row_spec(ROW_BLK, 1), _full_spec((4, D, D))]
        + [_full_spec((1, D))] * 3,
        out_specs=[_row_spec(ROW_BLK, D), _row_spec(ROW_BLK, D)],
        out_shape=[jax.ShapeDtypeStruct((N, D), jnp.float32)] * 2,
    )(h0, a1, a2, part3, part3, dis, wk, b.reshape(1, D), g.reshape(1, D),
      bt.reshape(1, D))


def _final(h0, a1, a2, part3, dis, wk, b, g, bt, wp1, bp1, wp2, bp2):
    return pl.pallas_call(
        _final_body,
        grid=_GRID,
        in_specs=[_row_spec(ROW_BLK, D)] * 3
        + [_part_spec(0), _part_spec(1)]
        + [_row_spec(ROW_BLK, 1), _full_spec((4, D, D))]
        + [_full_spec((1, D))] * 3
        + [_full_spec((D, D)), _full_spec((1, D)),
           _full_spec((D, 1)), _full_spec((1, 1))],
        out_specs=pl.BlockSpec((ROW_BLK, 1), lambda i: (i, 0)),
        out_shape=jax.ShapeDtypeStruct((N, 1), jnp.float32),
    )(h0, a1, a2, part3, part3, dis, wk, b.reshape(1, D), g.reshape(1, D),
      bt.reshape(1, D), wp1, bp1.reshape(1, D), wp2, bp2.reshape(1, 1))


def kernel(x, edge_index, W_pre, b_pre, W_mp1, b_mp1, g1, bt1,
           W_mp2, b_mp2, g2, bt2, W_po1, b_po1, W_po2, b_po2):
    dst = edge_index[1]

    degp = _sc_deg(dst)
    deg0 = degp[0, :N].reshape(N, 1)
    deg1 = degp[1, :N].reshape(N, 1)

    h0, u0, dis, dis2 = _pre(x, W_pre, b_pre, deg0, deg1)

    part1 = _sc_hop(u0, edge_index)
    a1, u1 = _hopcomb(part1, dis2)
    part2 = _sc_hop(u1, edge_index)
    a2, u2 = _hopcomb(part2, dis2)
    part3 = _sc_hop(u2, edge_index)

    h0b, u0b = _combine(h0, a1, a2, part3, dis, W_mp1, b_mp1, g1, bt1)

    part4 = _sc_hop(u0b, edge_index)
    b1, v1 = _hopcomb(part4, dis2)
    part5 = _sc_hop(v1, edge_index)
    b2, v2 = _hopcomb(part5, dis2)
    part6 = _sc_hop(v2, edge_index)

    return _final(h0b, b1, b2, part6, dis, W_mp2, b_mp2, g2, bt2,
                  W_po1, b_po1, W_po2, b_po2)


# speedup vs baseline: 1.0922x; 1.0922x over previous
import functools

import jax
import jax.numpy as jnp
from jax import lax
from jax.experimental import pallas as pl
from jax.experimental.pallas import tpu as pltpu
from jax.experimental.pallas import tpu_sc as plsc

N = 10000
E = 320000
D = 128
ROW_BLK = 1000

NW = 32
NSUB = 16
NPAD = 10240
TPR = NPAD // NSUB
GCHUNK = 128
NCH = E // GCHUNK
NBUF = 3
NBUFH = 2

_mesh = plsc.VectorSubcoreMesh(core_axis_name="c", subcore_axis_name="s")
_CP = pltpu.CompilerParams(needs_layout_passes=False)


def _ids():
    cid = lax.axis_index("c")
    sid = lax.axis_index("s")
    return cid, sid, sid * 2 + cid


def _ntrips(wid):
    return (NCH - wid + NW - 1) // NW


def _deg_body(dst_hbm, degp_hbm, sdeg, onesv, idxv, zb,
              isem, ssem):
    cid, sid, wid = _ids()
    ones = jnp.ones((16,), jnp.float32)
    zrow = jnp.zeros((16,), jnp.float32)
    for j in range(GCHUNK // 16):
        onesv[pl.ds(j * 16, 16)] = ones

    def _zb(i, _):
        zb[pl.ds(i * 16, 16)] = zrow
        return 0

    lax.fori_loop(0, TPR // 16, _zb, 0)
    pltpu.sync_copy(zb, sdeg.at[pl.ds(sid * TPR, TPR)])
    plsc.subcore_barrier()

    ntrips = _ntrips(wid)
    ngrp = (ntrips + NBUF - 1) // NBUF

    def _grp(g, _):
        base = g * NBUF
        for q in range(NBUF):
            t = base + q
            c = wid + t * NW

            @pl.when(t < ntrips)
            def _(q=q, t=t, c=c):
                @pl.when(t >= NBUF)
                def _():
                    pltpu.make_async_copy(onesv, sdeg.at[idxv.at[q]],
                                          ssem.at[q]).wait()
                pltpu.async_copy(dst_hbm.at[pl.ds(c * GCHUNK, GCHUNK)],
                                 idxv.at[q], isem.at[q])
        for q in range(NBUF):
            t = base + q

            @pl.when(t < ntrips)
            def _(q=q):
                pltpu.make_async_copy(dst_hbm.at[pl.ds(0, GCHUNK)],
                                      idxv.at[q], isem.at[q]).wait()
                pltpu.async_copy(onesv, sdeg.at[idxv.at[q]], ssem.at[q],
                                 add=True)
        return 0

    lax.fori_loop(0, ngrp, _grp, 0)
    for q in range(NBUF):
        @pl.when(q < ntrips)
        def _(q=q):
            pltpu.make_async_copy(onesv, sdeg.at[idxv.at[q]],
                                  ssem.at[q]).wait()
    plsc.subcore_barrier()

    pltpu.sync_copy(sdeg.at[pl.ds(sid * TPR, TPR)],
                    degp_hbm.at[cid, pl.ds(sid * TPR, TPR)])


def _sc_deg(dst):
    f = pl.kernel(
        _deg_body,
        compiler_params=_CP,
        out_type=jax.ShapeDtypeStruct((2, NPAD), jnp.float32),
        mesh=_mesh,
        scratch_types=[
            pltpu.VMEM_SHARED((NPAD,), jnp.float32),
            pltpu.VMEM((GCHUNK,), jnp.float32),
            pltpu.VMEM((NBUF, GCHUNK), jnp.int32),
            pltpu.VMEM((TPR,), jnp.float32),
            pltpu.SemaphoreType.DMA((NBUF,)),
            pltpu.SemaphoreType.DMA((NBUF,)),
        ],
    )
    return f(dst)


def _hop_body(u_hbm, ei_hbm, part_hbm,
              sacc, gbuf, eidx, zbuf, isem, gsem, ssem):
    cid, sid, wid = _ids()
    zrow = jnp.zeros((16,), jnp.float32)

    def _zb(i, _):
        for j in range(D // 16):
            zbuf[i, pl.ds(j * 16, 16)] = zrow
        return 0

    lax.fori_loop(0, 32, _zb, 0)
    for i in range(TPR // 32):
        pltpu.sync_copy(zbuf,
                        sacc.at[pl.ds(sid * TPR + i * 32, 32), :])
    plsc.subcore_barrier()

    ntrips = _ntrips(wid)
    NEI = 2 * NBUFH
    ngrp = (ntrips + NEI - 1) // NEI

    def _stage(t, e):
        c = wid + t * NW
        pltpu.async_copy(ei_hbm.at[:, pl.ds(c * GCHUNK, GCHUNK)],
                         eidx.at[e], isem.at[e])

    for e in range(NBUFH):
        @pl.when(e < ntrips)
        def _(e=e):
            _stage(jnp.int32(e), e)

    def _grp(g, _):
        base = g * NEI
        for pair in range(NEI // NBUFH):
            for q in range(NBUFH):
                qe = pair * NBUFH + q
                t = base + qe

                @pl.when(t < ntrips)
                def _(q=q, qe=qe, t=t):
                    @pl.when(t >= NBUFH)
                    def _():
                        pltpu.make_async_copy(gbuf.at[q],
                                              sacc.at[eidx.at[0, 1]],
                                              ssem.at[q]).wait()

                    @pl.when(t + NBUFH < ntrips)
                    def _():
                        _stage(t + NBUFH, (qe + NBUFH) % NEI)
            for q in range(NBUFH):
                qe = pair * NBUFH + q
                t = base + qe

                @pl.when(t < ntrips)
                def _(q=q, qe=qe):
                    pltpu.make_async_copy(ei_hbm.at[:, pl.ds(0, GCHUNK)],
                                          eidx.at[qe], isem.at[qe]).wait()
                    pltpu.async_copy(u_hbm.at[eidx.at[qe, 0]], gbuf.at[q],
                                     gsem.at[q])
            for q in range(NBUFH):
                qe = pair * NBUFH + q
                t = base + qe

                @pl.when(t < ntrips)
                def _(q=q, qe=qe):
                    pltpu.make_async_copy(u_hbm.at[eidx.at[qe, 0]],
                                          gbuf.at[q], gsem.at[q]).wait()
                    pltpu.async_copy(gbuf.at[q], sacc.at[eidx.at[qe, 1]],
                                     ssem.at[q], add=True)
        return 0

    lax.fori_loop(0, ngrp, _grp, 0)
    for q in range(NBUFH):
        @pl.when(q < ntrips)
        def _(q=q):
            pltpu.make_async_copy(gbuf.at[q], sacc.at[eidx.at[0, 1]],
                                  ssem.at[q]).wait()
    plsc.subcore_barrier()

    for i in range(TPR // GCHUNK):
        r = sid * TPR + i * GCHUNK
        pltpu.sync_copy(sacc.at[pl.ds(r, GCHUNK), :],
                        part_hbm.at[cid, pl.ds(r, GCHUNK), :])


def _sc_hop(u, ei):
    f = pl.kernel(
        _hop_body,
        compiler_params=_CP,
        out_type=jax.ShapeDtypeStruct((2, NPAD, D), jnp.float32),
        mesh=_mesh,
        scratch_types=[
            pltpu.VMEM_SHARED((NPAD, D), jnp.float32),
            pltpu.VMEM((NBUFH, GCHUNK, D), jnp.float32),
            pltpu.VMEM((2 * NBUFH, 2, GCHUNK), jnp.int32),
            pltpu.VMEM((32, D), jnp.float32),
            pltpu.SemaphoreType.DMA((2 * NBUFH,)),
            pltpu.SemaphoreType.DMA((NBUFH,)),
            pltpu.SemaphoreType.DMA((NBUFH,)),
        ],
    )
    return f(u, ei)


def _pre_body(x_ref, w_ref, b_ref, d0_ref, d1_ref,
              h_ref, u_ref, dis_ref, dis2_ref):
    deg = d0_ref[...] + d1_ref[...]
    dis = jnp.where(deg > 0, lax.rsqrt(jnp.maximum(deg, 1.0)), 0.0)
    dis_ref[...] = dis
    dis2_ref[...] = dis * dis
    h = jnp.maximum(
        jnp.dot(x_ref[...], w_ref[...], preferred_element_type=jnp.float32)
        + b_ref[...], 0.0)
    h_ref[...] = h
    u_ref[...] = h * dis


def _hopcomb_body(p0_ref, p1_ref, dis2_ref, a_ref, u_ref):
    s = p0_ref[0] + p1_ref[0]
    a_ref[...] = s
    u_ref[...] = s * dis2_ref[...]


def _ln(acc, g, bt):
    mu = jnp.mean(acc, axis=-1, keepdims=True)
    var = jnp.mean((acc - mu) ** 2, axis=-1, keepdims=True)
    return (acc - mu) * lax.rsqrt(var + 1e-5) * g + bt


def _combine_body(h0_ref, a1_ref, a2_ref, p30_ref, p31_ref, dis_ref, wk_ref,
                  b_ref, g_ref, bt_ref, h_ref, u_ref):
    d = dis_ref[...]
    acc = jnp.dot(h0_ref[...], wk_ref[0], preferred_element_type=jnp.float32)
    acc += jnp.dot(a1_ref[...] * d, wk_ref[1], preferred_element_type=jnp.float32)
    acc += jnp.dot(a2_ref[...] * d, wk_ref[2], preferred_element_type=jnp.float32)
    a3 = p30_ref[0] + p31_ref[0]
    acc += jnp.dot(a3 * d, wk_ref[3], preferred_element_type=jnp.float32)
    acc = jnp.maximum(acc + b_ref[...], 0.0)
    y = _ln(acc, g_ref[...], bt_ref[...])
    h_ref[...] = y
    u_ref[...] = y * d


def _final_body(h0_ref, a1_ref, a2_ref, p30_ref, p31_ref, dis_ref, wk_ref,
                b_ref, g_ref, bt_ref, wp1_ref, bp1_ref, wp2_ref, bp2_ref,
                o_ref):
    d = dis_ref[...]
    acc = jnp.dot(h0_ref[...], wk_ref[0], preferred_element_type=jnp.float32)
    acc += jnp.dot(a1_ref[...] * d, wk_ref[1], preferred_element_type=jnp.float32)
    acc += jnp.dot(a2_ref[...] * d, wk_ref[2], preferred_element_type=jnp.float32)
    a3 = p30_ref[0] + p31_ref[0]
    acc += jnp.dot(a3 * d, wk_ref[3], preferred_element_type=jnp.float32)
    acc = jnp.maximum(acc + b_ref[...], 0.0)
    y = _ln(acc, g_ref[...], bt_ref[...])
    z = jnp.maximum(
        jnp.dot(y, wp1_ref[...], preferred_element_type=jnp.float32)
        + bp1_ref[...], 0.0)
    o_ref[...] = jnp.dot(z, wp2_ref[...], preferred_element_type=jnp.float32) \
        + bp2_ref[...]


def _row_spec(blk, width):
    return pl.BlockSpec((blk, width), lambda i: (i, 0))


def _part_spec(core):
    return pl.BlockSpec((1, ROW_BLK, D), lambda i, core=core: (core, i, 0))


def _full_spec(shape):
    nd = len(shape)
    return pl.BlockSpec(shape, lambda i: (0,) * nd)


_GRID = (N // ROW_BLK,)


def _pre(x, w, b, deg0, deg1):
    return pl.pallas_call(
        _pre_body,
        grid=_GRID,
        in_specs=[_row_spec(ROW_BLK, D), _full_spec((D, D)),
                  _full_spec((1, D)), _row_spec(ROW_BLK, 1),
                  _row_spec(ROW_BLK, 1)],
        out_specs=[_row_spec(ROW_BLK, D), _row_spec(ROW_BLK, D),
                   _row_spec(ROW_BLK, 1), _row_spec(ROW_BLK, 1)],
        out_shape=[jax.ShapeDtypeStruct((N, D), jnp.float32),
                   jax.ShapeDtypeStruct((N, D), jnp.float32),
                   jax.ShapeDtypeStruct((N, 1), jnp.float32),
                   jax.ShapeDtypeStruct((N, 1), jnp.float32)],
    )(x, w, b.reshape(1, D), deg0, deg1)


def _hopcomb(part, dis2):
    return pl.pallas_call(
        _hopcomb_body,
        grid=_GRID,
        in_specs=[_part_spec(0), _part_spec(1), _row_spec(ROW_BLK, 1)],
        out_specs=[_row_spec(ROW_BLK, D), _row_spec(ROW_BLK, D)],
        out_shape=[jax.ShapeDtypeStruct((N, D), jnp.float32)] * 2,
    )(part, part, dis2)


def _combine(h0, a1, a2, part3, dis, wk, b, g, bt):
    return pl.pallas_call(
        _combine_body,
        grid=_GRID,
        in_specs=[_row_spec(ROW_BLK, D)] * 3
        + [_part_spec(0), _part_spec(1)]
        + [_row_spec(ROW_BLK, 1), _full_spec((4, D, D))]
        + [_full_spec((1, D))] * 3,
        out_specs=[_row_spec(ROW_BLK, D), _row_spec(ROW_BLK, D)],
        out_shape=[jax.ShapeDtypeStruct((N, D), jnp.float32)] * 2,
    )(h0, a1, a2, part3, part3, dis, wk, b.reshape(1, D), g.reshape(1, D),
      bt.reshape(1, D))


def _final(h0, a1, a2, part3, dis, wk, b, g, bt, wp1, bp1, wp2, bp2):
    return pl.pallas_call(
        _final_body,
        grid=_GRID,
        in_specs=[_row_spec(ROW_BLK, D)] * 3
        + [_part_spec(0), _part_spec(1)]
        + [_row_spec(ROW_BLK, 1), _full_spec((4, D, D))]
        + [_full_spec((1, D))] * 3
        + [_full_spec((D, D)), _full_spec((1, D)),
           _full_spec((D, 1)), _full_spec((1, 1))],
        out_specs=pl.BlockSpec((ROW_BLK, 1), lambda i: (i, 0)),
        out_shape=jax.ShapeDtypeStruct((N, 1), jnp.float32),
    )(h0, a1, a2, part3, part3, dis, wk, b.reshape(1, D), g.reshape(1, D),
      bt.reshape(1, D), wp1, bp1.reshape(1, D), wp2, bp2.reshape(1, 1))


def kernel(x, edge_index, W_pre, b_pre, W_mp1, b_mp1, g1, bt1,
           W_mp2, b_mp2, g2, bt2, W_po1, b_po1, W_po2, b_po2):
    dst = edge_index[1]

    degp = _sc_deg(dst)
    deg0 = degp[0, :N].reshape(N, 1)
    deg1 = degp[1, :N].reshape(N, 1)

    h0, u0, dis, dis2 = _pre(x, W_pre, b_pre, deg0, deg1)

    part1 = _sc_hop(u0, edge_index)
    a1, u1 = _hopcomb(part1, dis2)
    part2 = _sc_hop(u1, edge_index)
    a2, u2 = _hopcomb(part2, dis2)
    part3 = _sc_hop(u2, edge_index)

    h0b, u0b = _combine(h0, a1, a2, part3, dis, W_mp1, b_mp1, g1, bt1)

    part4 = _sc_hop(u0b, edge_index)
    b1, v1 = _hopcomb(part4, dis2)
    part5 = _sc_hop(v1, edge_index)
    b2, v2 = _hopcomb(part5, dis2)
    part6 = _sc_hop(v2, edge_index)

    return _final(h0b, b1, b2, part6, dis, W_mp2, b_mp2, g2, bt2,
                  W_po1, b_po1, W_po2, b_po2)
